# Initial kernel scaffold; baseline (speedup 1.0000x reference)
import jax
import jax.numpy as jnp
from jax.experimental import pallas as pl


def kernel(x, ln1_w, ln2_w, wq, wk, wv, wo, gate_w, w1, w2, w3):
    raise NotImplementedError("write your pallas kernel here")


# speedup vs baseline: 1.0213x; 1.0213x over previous
import functools
import math

import jax
import jax.numpy as jnp
from jax import lax
from jax.experimental import pallas as pl
from jax.experimental.pallas import tpu as pltpu

EPS = 1e-5
NEG = -1e30


def _qkv_body(x_ref, w_ref, out_ref):
    xb = x_ref[...]
    r = lax.rsqrt(jnp.mean(xb * xb, axis=1, keepdims=True) + EPS)
    out_ref[...] = jnp.dot(xb * r, w_ref[...], preferred_element_type=jnp.float32)


def _qkv(x2d, wqkv, bm=256, bn=512):
    S, H = x2d.shape
    N = wqkv.shape[1]
    return pl.pallas_call(
        _qkv_body,
        grid=(S // bm, N // bn),
        in_specs=[
            pl.BlockSpec((bm, H), lambda i, j: (i, 0)),
            pl.BlockSpec((H, bn), lambda i, j: (0, j)),
        ],
        out_specs=pl.BlockSpec((bm, bn), lambda i, j: (i, j)),
        out_shape=jax.ShapeDtypeStruct((S, N), jnp.float32),
    )(x2d, wqkv)


def _attn_body(q_ref, k_ref, v_ref, o_ref, *, scale):
    q = q_ref[...]
    k = k_ref[...]
    s = lax.dot_general(q, k, (((1,), (1,)), ((), ())),
                        preferred_element_type=jnp.float32) * scale
    m = jnp.max(s, axis=1, keepdims=True)
    p = jnp.exp(s - m)
    l = jnp.sum(p, axis=1, keepdims=True)
    o_ref[...] = jnp.dot(p, v_ref[...], preferred_element_type=jnp.float32) / l


def _attention(qkv, S, NH, DH, bq=512):
    H = NH * DH
    return pl.pallas_call(
        functools.partial(_attn_body, scale=1.0 / math.sqrt(DH)),
        grid=(NH, S // bq),
        in_specs=[
            pl.BlockSpec((bq, DH), lambda h, i: (i, h)),
            pl.BlockSpec((S, DH), lambda h, i: (0, NH + h)),
            pl.BlockSpec((S, DH), lambda h, i: (0, 2 * NH + h)),
        ],
        out_specs=pl.BlockSpec((bq, DH), lambda h, i: (i, h)),
        out_shape=jax.ShapeDtypeStruct((S, H), jnp.float32),
    )(qkv, qkv, qkv)


def _proj_router_body(o_ref, x_ref, wo_ref, ln2_ref, gw_ref,
                      x2_ref, h2_ref, i1_ref, i2_ref, w1_ref, w2_ref,
                      we_ref):
    x2 = x_ref[...] + jnp.dot(o_ref[...], wo_ref[...],
                              preferred_element_type=jnp.float32)
    x2_ref[...] = x2
    r = lax.rsqrt(jnp.mean(x2 * x2, axis=1, keepdims=True) + EPS)
    h2 = x2 * r * ln2_ref[...][None, :]
    h2_ref[...] = h2
    lg = jnp.dot(h2, gw_ref[...], preferred_element_type=jnp.float32)
    bm, ncol = lg.shape
    col = lax.broadcasted_iota(jnp.int32, (bm, ncol), 1)
    lg = jnp.where(col < 8, lg, NEG)
    m1 = jnp.max(lg, axis=1)
    i1 = jnp.min(jnp.where(lg == m1[:, None], col, ncol), axis=1)
    lg2 = jnp.where(col == i1[:, None], NEG, lg)
    m2 = jnp.max(lg2, axis=1)
    i2 = jnp.min(jnp.where(lg2 == m2[:, None], col, ncol), axis=1)
    e2 = jnp.exp(m2 - m1)
    w1 = 1.0 / (1.0 + e2)
    w2 = e2 / (1.0 + e2)
    i1_ref[...] = i1
    i2_ref[...] = i2
    w1_ref[...] = w1
    w2_ref[...] = w2
    ecol = lax.broadcasted_iota(jnp.int32, (bm, 8), 1)
    we_ref[...] = (jnp.where(ecol == i1[:, None], w1[:, None], 0.0)
                   + jnp.where(ecol == i2[:, None], w2[:, None], 0.0))


def _proj_router(o, x2d, wo, ln2_w, gate_wp, bm=256):
    S, H = x2d.shape
    outs = pl.pallas_call(
        _proj_router_body,
        grid=(S // bm,),
        in_specs=[
            pl.BlockSpec((bm, H), lambda i: (i, 0)),
            pl.BlockSpec((bm, H), lambda i: (i, 0)),
            pl.BlockSpec((H, H), lambda i: (0, 0)),
            pl.BlockSpec((H,), lambda i: (0,)),
            pl.BlockSpec((H, 128), lambda i: (0, 0)),
        ],
        out_specs=[
            pl.BlockSpec((bm, H), lambda i: (i, 0)),
            pl.BlockSpec((bm, H), lambda i: (i, 0)),
            pl.BlockSpec((bm,), lambda i: (i,)),
            pl.BlockSpec((bm,), lambda i: (i,)),
            pl.BlockSpec((bm,), lambda i: (i,)),
            pl.BlockSpec((bm,), lambda i: (i,)),
            pl.BlockSpec((bm, 8), lambda i: (i, 0)),
        ],
        out_shape=[
            jax.ShapeDtypeStruct((S, H), jnp.float32),
            jax.ShapeDtypeStruct((S, H), jnp.float32),
            jax.ShapeDtypeStruct((S,), jnp.int32),
            jax.ShapeDtypeStruct((S,), jnp.int32),
            jax.ShapeDtypeStruct((S,), jnp.float32),
            jax.ShapeDtypeStruct((S,), jnp.float32),
            jax.ShapeDtypeStruct((S, 8), jnp.float32),
        ],
    )(o, x2d, wo, ln2_w, gate_wp)
    return outs


def _moe_dense_body(we_ref, h2_ref, x2_ref, w1_ref, w3_ref, w2_ref, out_ref):
    e = pl.program_id(1)
    f = pl.program_id(2)

    @pl.when(jnp.logical_and(e == 0, f == 0))
    def _():
        out_ref[...] = x2_ref[...]

    h2 = h2_ref[...]
    g = jnp.dot(h2, w1_ref[0], preferred_element_type=jnp.float32)
    u = jnp.dot(h2, w3_ref[0], preferred_element_type=jnp.float32)
    hid = (g / (1.0 + jnp.exp(-g))) * u
    out_ref[...] += (jnp.dot(hid, w2_ref[0], preferred_element_type=jnp.float32)
                     * we_ref[0, 0, :][:, None])


def _moe_dense(we_arr, h2, x2, w1, w3, w2, bm=512, bf=256):
    S, H = h2.shape
    E, _, FF = w1.shape
    return pl.pallas_call(
        _moe_dense_body,
        grid=(S // bm, E, FF // bf),
        in_specs=[
            pl.BlockSpec((1, 1, bm), lambda i, e, f: (e, 0, i)),
            pl.BlockSpec((bm, H), lambda i, e, f: (i, 0)),
            pl.BlockSpec((bm, H), lambda i, e, f: (i, 0)),
            pl.BlockSpec((1, H, bf), lambda i, e, f: (e, 0, f)),
            pl.BlockSpec((1, H, bf), lambda i, e, f: (e, 0, f)),
            pl.BlockSpec((1, bf, H), lambda i, e, f: (e, f, 0)),
        ],
        out_specs=pl.BlockSpec((bm, H), lambda i, e, f: (i, 0)),
        out_shape=jax.ShapeDtypeStruct((S, H), jnp.float32),
    )(we_arr, h2, x2, w1, w3, w2)


def kernel(x, ln1_w, ln2_w, wq, wk, wv, wo, gate_w, w1, w2, w3):
    B, S, H = x.shape
    NH, DH = 16, H // 16
    x2d = x.reshape(S, H)
    wqkv = jnp.concatenate([wq, wk, wv], axis=1) * ln1_w[:, None]
    gate_wp = jnp.pad(gate_w, ((0, 0), (0, 128 - gate_w.shape[1])))

    qkv = _qkv(x2d, wqkv)
    o = _attention(qkv, S, NH, DH)
    x2, h2, i1, i2, rw1, rw2, we = _proj_router(o, x2d, wo, ln2_w, gate_wp)
    we_arr = we.T.reshape(8, 1, S)
    out = _moe_dense(we_arr, h2, x2, w1, w3, w2)
    return out.reshape(B, S, H)


# speedup vs baseline: 1.2816x; 1.2549x over previous
import functools
import math

import jax
import jax.numpy as jnp
from jax import lax
from jax.experimental import pallas as pl
from jax.experimental.pallas import tpu as pltpu
from jax.experimental.pallas import tpu_sc as plsc

EPS = 1e-5
NEG = -1e30


def _dot6(a, b, dn=None):
    def split(x):
        x1 = x.astype(jnp.bfloat16)
        r = x - x1.astype(jnp.float32)
        x2 = r.astype(jnp.bfloat16)
        x3 = (r - x2.astype(jnp.float32)).astype(jnp.bfloat16)
        return x1, x2, x3
    a1, a2, a3 = split(a)
    b1, b2, b3 = split(b)
    if dn is None:
        dn = (((1,), (0,)), ((), ()))
    d = lambda u, v: lax.dot_general(u, v, dn, preferred_element_type=jnp.float32)
    return ((d(a3, b1) + d(a2, b2) + d(a1, b3))
            + (d(a2, b1) + d(a1, b2))) + d(a1, b1)
BS = 256


def _qkv_body(x_ref, w_ref, out_ref):
    xb = x_ref[...]
    r = lax.rsqrt(jnp.mean(xb * xb, axis=1, keepdims=True) + EPS)
    out_ref[...] = jnp.dot(xb * r, w_ref[...], preferred_element_type=jnp.float32)


def _qkv(x2d, wqkv, bm=256, bn=512):
    S, H = x2d.shape
    N = wqkv.shape[1]
    return pl.pallas_call(
        _qkv_body,
        grid=(S // bm, N // bn),
        in_specs=[
            pl.BlockSpec((bm, H), lambda i, j: (i, 0)),
            pl.BlockSpec((H, bn), lambda i, j: (0, j)),
        ],
        out_specs=pl.BlockSpec((bm, bn), lambda i, j: (i, j)),
        out_shape=jax.ShapeDtypeStruct((S, N), jnp.float32),
    )(x2d, wqkv)


def _attn_body(q_ref, k_ref, v_ref, o_ref, *, scale):
    q = q_ref[...]
    k = k_ref[...]
    s = lax.dot_general(q, k, (((1,), (1,)), ((), ())),
                        preferred_element_type=jnp.float32) / scale
    m = jnp.max(s, axis=1, keepdims=True)
    p = jnp.exp(s - m)
    p = p / jnp.sum(p, axis=1, keepdims=True)
    o_ref[...] = jnp.dot(p, v_ref[...], preferred_element_type=jnp.float32)


def _attention(qkv, S, NH, DH, bq=512):
    H = NH * DH
    return pl.pallas_call(
        functools.partial(_attn_body, scale=math.sqrt(DH)),
        grid=(NH, S // bq),
        in_specs=[
            pl.BlockSpec((bq, DH), lambda h, i: (i, h)),
            pl.BlockSpec((S, DH), lambda h, i: (0, NH + h)),
            pl.BlockSpec((S, DH), lambda h, i: (0, 2 * NH + h)),
        ],
        out_specs=pl.BlockSpec((bq, DH), lambda h, i: (i, h)),
        out_shape=jax.ShapeDtypeStruct((S, H), jnp.float32),
    )(qkv, qkv, qkv)


def _proj_router_body(o_ref, x_ref, wo_ref, ln2_ref, gw_ref,
                      x2_ref, h2_ref, i1_ref, i2_ref, w1_ref, w2_ref):
    x2 = x_ref[...] + jnp.dot(o_ref[...], wo_ref[...],
                              preferred_element_type=jnp.float32)
    x2_ref[...] = x2
    r = lax.rsqrt(jnp.mean(x2 * x2, axis=1, keepdims=True) + EPS)
    h2 = x2 * r * ln2_ref[...][None, :]
    h2_ref[...] = h2
    lg = jnp.dot(h2, gw_ref[...], preferred_element_type=jnp.float32)
    bm, ncol = lg.shape
    col = lax.broadcasted_iota(jnp.int32, (bm, ncol), 1)
    lg = jnp.where(col < 8, lg, NEG)
    m1 = jnp.max(lg, axis=1)
    i1 = jnp.min(jnp.where(lg == m1[:, None], col, ncol), axis=1)
    lg2 = jnp.where(col == i1[:, None], NEG, lg)
    m2 = jnp.max(lg2, axis=1)
    i2 = jnp.min(jnp.where(lg2 == m2[:, None], col, ncol), axis=1)
    e2 = jnp.exp(m2 - m1)
    i1_ref[...] = i1
    i2_ref[...] = i2
    w1_ref[...] = 1.0 / (1.0 + e2)
    w2_ref[...] = e2 / (1.0 + e2)


def _proj_router(o, x2d, wo, ln2_w, gate_wp, bm=256):
    S, H = x2d.shape
    return pl.pallas_call(
        _proj_router_body,
        grid=(S // bm,),
        in_specs=[
            pl.BlockSpec((bm, H), lambda i: (i, 0)),
            pl.BlockSpec((bm, H), lambda i: (i, 0)),
            pl.BlockSpec((H, H), lambda i: (0, 0)),
            pl.BlockSpec((H,), lambda i: (0,)),
            pl.BlockSpec((H, 128), lambda i: (0, 0)),
        ],
        out_specs=[
            pl.BlockSpec((bm, H), lambda i: (i, 0)),
            pl.BlockSpec((bm, H), lambda i: (i, 0)),
            pl.BlockSpec((bm,), lambda i: (i,)),
            pl.BlockSpec((bm,), lambda i: (i,)),
            pl.BlockSpec((bm,), lambda i: (i,)),
            pl.BlockSpec((bm,), lambda i: (i,)),
        ],
        out_shape=[
            jax.ShapeDtypeStruct((S, H), jnp.float32),
            jax.ShapeDtypeStruct((S, H), jnp.float32),
            jax.ShapeDtypeStruct((S,), jnp.int32),
            jax.ShapeDtypeStruct((S,), jnp.int32),
            jax.ShapeDtypeStruct((S,), jnp.float32),
            jax.ShapeDtypeStruct((S,), jnp.float32),
        ],
    )(o, x2d, wo, ln2_w, gate_wp)


def _make_dispatch(S, E, G, NTP):
    NPAIR = 2 * S
    NCH = NPAIR // 16
    mesh = plsc.VectorSubcoreMesh(core_axis_name="c", subcore_axis_name="s", num_cores=2, num_subcores=16)

    @functools.partial(
        pl.kernel,
        mesh=mesh,
        compiler_params=pltpu.CompilerParams(needs_layout_passes=False),
        out_type=[
            jax.ShapeDtypeStruct((G,), jnp.int32),
            jax.ShapeDtypeStruct((NPAIR,), jnp.int32),
            jax.ShapeDtypeStruct((NTP,), jnp.int32),
        ],
        scratch_types=[
            pltpu.VMEM((NPAIR,), jnp.int32),
            pltpu.VMEM((G,), jnp.int32),
            pltpu.VMEM((NPAIR,), jnp.int32),
            pltpu.VMEM((16,), jnp.int32),
            pltpu.VMEM((E, 16), jnp.int32),
            pltpu.VMEM((E * NPAIR,), jnp.int32),
            pltpu.VMEM((32,), jnp.int32),
            pltpu.VMEM_SHARED((E, 16), jnp.int32),
            pltpu.VMEM_SHARED((E * NPAIR,), jnp.int32),
        ],
    )
    def dispatch(i1_hbm, i2_hbm, perm_hbm, pos_hbm, te_hbm,
                 ids_v, perm_v, pos_v, cnt_v, cnt_all_v, big_v, te_v,
                 cnt_sh, pos_sh):
        c = lax.axis_index("c")
        s = lax.axis_index("s")
        on0 = c == 0
        is_worker = jnp.logical_and(on0, s < E)
        lane = lax.iota(jnp.int32, 16)
        zeros16 = jnp.zeros((16,), jnp.int32)

        @pl.when(is_worker)
        def _():
            def zb(q, carry):
                pos_v[pl.ds(q * 16, 16)] = zeros16
                return carry
            lax.fori_loop(0, NCH, zb, 0)

        @pl.when(is_worker)
        def _():
            pltpu.sync_copy(i1_hbm, ids_v.at[pl.ds(0, S)])
            pltpu.sync_copy(i2_hbm, ids_v.at[pl.ds(S, S)])

            def cb(q, cnt):
                chunk = ids_v[pl.ds(q * 16, 16)]
                cs = plsc.cumsum((chunk == s).astype(jnp.int32))
                return cnt + jnp.max(cs)
            cnt = lax.fori_loop(0, NCH, cb, jnp.int32(0))
            cnt_v[...] = zeros16 + cnt
            pltpu.sync_copy(cnt_v, cnt_sh.at[s])

        plsc.subcore_barrier()

        @pl.when(jnp.logical_and(on0, s < 12))
        def _():
            pltpu.sync_copy(cnt_sh, cnt_all_v)

        cnts = [cnt_all_v[j][0] for j in range(E)]
        padded = [((cj + BS - 1) // BS) * BS for cj in cnts]
        goffs = [jnp.int32(0)]
        for j in range(E):
            goffs.append(goffs[-1] + padded[j])
        total = goffs[E]
        goff_e = jnp.int32(0)
        padded_e = jnp.int32(0)
        for j in range(E):
            goff_e = jnp.where(s == j, goffs[j], goff_e)
            padded_e = jnp.where(s == j, padded[j], padded_e)

        @pl.when(is_worker)
        def _():
            def zp(q, carry):
                perm_v[pl.ds(q * 16, 16)] = zeros16
                return carry
            lax.fori_loop(0, G // 16, zp, 0)

            def sb(q, run):
                chunk = ids_v[pl.ds(q * 16, 16)]
                mask = chunk == s
                cs = plsc.cumsum(mask.astype(jnp.int32))
                slots = run + cs - 1
                pair = lane + q * 16
                tok = pair & (S - 1)
                plsc.store_scatter(perm_v, [slots], tok, mask=mask)
                pos_v[pl.ds(q * 16, 16)] = jnp.where(mask, slots, 0)
                return run + jnp.max(cs)
            lax.fori_loop(0, NCH, sb, jnp.zeros((16,), jnp.int32) + goff_e)

            pltpu.sync_copy(pos_v, pos_sh.at[pl.ds(pl.multiple_of(s * NPAIR, NPAIR), NPAIR)])

            def ob(q, carry):
                off = pl.multiple_of(goff_e + q * BS, BS)
                pltpu.sync_copy(perm_v.at[pl.ds(off, BS)],
                                perm_hbm.at[pl.ds(off, BS)])
                return carry
            lax.fori_loop(0, padded_e // BS, ob, 0)

        plsc.subcore_barrier()

        @pl.when(jnp.logical_and(on0, s == E))
        def _():
            def zt(q, carry):
                perm_v[pl.ds(q * 16, 16)] = zeros16
                return carry
            lax.fori_loop(0, BS // 16, zt, 0)

            def tb(q, carry):
                off = pl.multiple_of(q * BS, BS)
                pltpu.sync_copy(perm_v.at[pl.ds(0, BS)],
                                perm_hbm.at[pl.ds(off, BS)])
                return carry
            lax.fori_loop(total // BS, G // BS, tb, 0)

        @pl.when(jnp.logical_and(on0, s == 9))
        def _():
            pltpu.sync_copy(pos_sh, big_v)

            def mb(q, carry):
                acc = zeros16
                for j in range(E):
                    acc = acc + big_v[pl.ds(j * NPAIR + q * 16, 16)]
                pos_v[pl.ds(q * 16, 16)] = acc
                return carry
            lax.fori_loop(0, NCH, mb, 0)
            pltpu.sync_copy(pos_v, pos_hbm)

        @pl.when(jnp.logical_and(on0, s == 10))
        def _():
            for c2 in range(NTP // 16):
                tid = lane + c2 * 16
                acc = jnp.zeros((16,), jnp.int32)
                for j in range(E):
                    acc = acc + (tid * BS >= goffs[j] + padded[j]).astype(jnp.int32)
                te_v[pl.ds(c2 * 16, 16)] = jnp.minimum(acc, E - 1)
            pltpu.sync_copy(te_v.at[pl.ds(0, NTP)], te_hbm)

    return dispatch


def _make_gather(N, Hd, M, CH=16):
    mesh = plsc.VectorSubcoreMesh(core_axis_name="c", subcore_axis_name="s", num_cores=2, num_subcores=16)
    NW = 32
    per = M // NW
    nch = per // CH

    @functools.partial(
        pl.kernel,
        mesh=mesh,
        compiler_params=pltpu.CompilerParams(needs_layout_passes=False),
        out_type=jax.ShapeDtypeStruct((M, Hd), jnp.float32),
        scratch_types=[
            pltpu.VMEM((CH,), jnp.int32),
            pltpu.VMEM((CH, Hd), jnp.float32),
            pltpu.SemaphoreType.DMA,
        ],
    )
    def gk(src_hbm, idx_hbm, out_hbm, idx_v, rows_v, sem):
        wid = lax.axis_index("s") * 2 + lax.axis_index("c")
        base = wid * per
        for ch in range(nch):
            o = base + ch * CH
            pltpu.sync_copy(idx_hbm.at[pl.ds(o, CH)], idx_v)
            pltpu.async_copy(src_hbm.at[idx_v], rows_v, sem).wait()
            pltpu.sync_copy(rows_v, out_hbm.at[pl.ds(o, CH)])

    return gk


def _hid_body(te_ref, xs_ref, w1_ref, w3_ref, out_ref):
    g = jnp.dot(xs_ref[...], w1_ref[0], preferred_element_type=jnp.float32)
    u = jnp.dot(xs_ref[...], w3_ref[0], preferred_element_type=jnp.float32)
    out_ref[...] = (g / (1.0 + jnp.exp(-g))) * u


def _moe_hid(te, xs, w1, w3, bf=1024):
    G, Hd = xs.shape
    E, _, FF = w1.shape
    NT = G // BS
    grid_spec = pltpu.PrefetchScalarGridSpec(
        num_scalar_prefetch=1,
        grid=(FF // bf, NT),
        in_specs=[
            pl.BlockSpec((BS, Hd), lambda f, i, te: (i, 0)),
            pl.BlockSpec((1, Hd, bf), lambda f, i, te: (te[i], 0, f)),
            pl.BlockSpec((1, Hd, bf), lambda f, i, te: (te[i], 0, f)),
        ],
        out_specs=pl.BlockSpec((BS, bf), lambda f, i, te: (i, f)),
    )
    return pl.pallas_call(
        _hid_body,
        grid_spec=grid_spec,
        out_shape=jax.ShapeDtypeStruct((G, FF), jnp.float32),
    )(te, xs, w1, w3)


def _y_body(te_ref, hid_ref, w2_ref, out_ref):
    out_ref[...] = jnp.dot(hid_ref[...], w2_ref[0],
                           preferred_element_type=jnp.float32)


def _moe_y(te, hid, w2, bh=1024):
    G, FF = hid.shape
    E, _, Hd = w2.shape
    NT = G // BS
    grid_spec = pltpu.PrefetchScalarGridSpec(
        num_scalar_prefetch=1,
        grid=(Hd // bh, NT),
        in_specs=[
            pl.BlockSpec((BS, FF), lambda h, i, te: (i, 0)),
            pl.BlockSpec((1, FF, bh), lambda h, i, te: (te[i], 0, h)),
        ],
        out_specs=pl.BlockSpec((BS, bh), lambda h, i, te: (i, h)),
    )
    return pl.pallas_call(
        _y_body,
        grid_spec=grid_spec,
        out_shape=jax.ShapeDtypeStruct((G, Hd), jnp.float32),
    )(te, hid, w2)


def _combine_body(x2_ref, y1_ref, y2_ref, r1_ref, r2_ref, out_ref):
    out_ref[...] = (x2_ref[...] + r1_ref[...] * y1_ref[...]
                    + r2_ref[...] * y2_ref[...])


def _combine(x2, y12, rw1, rw2, bm=512):
    S, H = x2.shape
    return pl.pallas_call(
        _combine_body,
        grid=(S // bm,),
        in_specs=[
            pl.BlockSpec((bm, H), lambda i: (i, 0)),
            pl.BlockSpec((bm, H), lambda i: (i, 0)),
            pl.BlockSpec((bm, H), lambda i: (S // bm + i, 0)),
            pl.BlockSpec((bm, 1), lambda i: (i, 0)),
            pl.BlockSpec((bm, 1), lambda i: (i, 0)),
        ],
        out_specs=pl.BlockSpec((bm, H), lambda i: (i, 0)),
        out_shape=jax.ShapeDtypeStruct((S, H), jnp.float32),
    )(x2, y12, y12, rw1, rw2)


def kernel(x, ln1_w, ln2_w, wq, wk, wv, wo, gate_w, w1, w2, w3):
    B, S, H = x.shape
    NH, DH = 16, H // 16
    E = w1.shape[0]
    G = 2 * S + E * BS
    NTP = 32
    x2d = x.reshape(S, H)
    wqkv = jnp.concatenate([wq, wk, wv], axis=1) * ln1_w[:, None]
    gate_wp = jnp.pad(gate_w, ((0, 0), (0, 128 - gate_w.shape[1])))

    qkv = _qkv(x2d, wqkv)
    o = _attention(qkv, S, NH, DH)
    x2, h2, i1, i2, rw1, rw2 = _proj_router(o, x2d, wo, ln2_w, gate_wp)

    perm, pos, te = _make_dispatch(S, E, G, NTP)(i1, i2)
    xs = _make_gather(S, H, G)(h2, perm)
    hid = _moe_hid(te, xs, w1, w3)
    y = _moe_y(te, hid, w2)
    y12 = _make_gather(G, H, 2 * S)(y, pos)
    out = _combine(x2, y12, rw1.reshape(S, 1), rw2.reshape(S, 1))
    return out.reshape(B, S, H)
